# baseline (speedup 1.0000x reference)
import functools

import jax
import jax.numpy as jnp
from jax import lax
from jax.experimental import pallas as pl
from jax.experimental.pallas import tpu as pltpu
from jax.experimental.pallas import tpu_sc as plsc

_L = 16
_NB = 4


def _sc_body(B, T, S, D, TPW, TC, x_hbm, emb_hbm, out_hbm, *scratch):
    wid = lax.axis_index("s") * 2 + lax.axis_index("c")
    t0 = wid * TPW
    cpb = TPW // TC
    nch = B * cpb
    shift = cpb.bit_length() - 1
    xbs, ebs = scratch[0:_NB], scratch[_NB:2 * _NB]
    sxs, ses, sos = (scratch[2 * _NB:3 * _NB], scratch[3 * _NB:4 * _NB],
                     scratch[4 * _NB:5 * _NB])

    def coords(i):
        b = lax.shift_right_logical(i, shift)
        c = lax.bitwise_and(i, cpb - 1)
        return b, t0 + c * TC

    def start_in(i, slot):
        b, tb = coords(i)
        pltpu.make_async_copy(
            x_hbm.at[b, pl.ds(tb, TC)], xbs[slot], sxs[slot]).start()
        pltpu.make_async_copy(
            emb_hbm.at[pl.ds(tb, TC)], ebs[slot], ses[slot]).start()

    def wait_in(slot):
        pltpu.make_async_copy(
            x_hbm.at[0, pl.ds(0, TC)], xbs[slot], sxs[slot]).wait()
        pltpu.make_async_copy(
            emb_hbm.at[pl.ds(0, TC)], ebs[slot], ses[slot]).wait()

    def start_out(i, slot):
        b, tb = coords(i)
        pltpu.make_async_copy(
            xbs[slot], out_hbm.at[b, pl.ds(tb, TC)], sos[slot]).start()

    def wait_out(slot):
        pltpu.make_async_copy(
            xbs[slot], out_hbm.at[0, pl.ds(0, TC)], sos[slot]).wait()

    def compute(slot):
        xb, eb = xbs[slot], ebs[slot]
        for t in range(TC):
            def body(k2, carry):
                for u in range(2):
                    sl = pl.ds(k2 * 2 * _L + u * _L, _L)
                    ev = eb[t, sl]
                    for s_ in range(S):
                        xb[t, s_, sl] = xb[t, s_, sl] + ev
                return carry
            lax.fori_loop(0, D // (2 * _L), body, 0)

    for j in range(_NB - 1):
        start_in(jnp.int32(j), j)

    def ring(iq, carry):
        for j in range(_NB):
            i = iq * _NB + j
            tgt = (j + _NB - 1) % _NB

            @pl.when(i + _NB - 1 < nch)
            def _prefetch():
                @pl.when(i >= 1)
                def _drain():
                    wait_out(tgt)
                start_in(i + _NB - 1, tgt)

            wait_in(j)
            compute(j)
            start_out(i, j)
        return carry

    lax.fori_loop(0, nch // _NB, ring, 0)
    for j in range(_NB):
        wait_out(j)


def kernel(x, pos_embedding):
    B, T, S, D = x.shape
    NW = 32
    TPW = T // NW
    TC = 4

    mesh = plsc.VectorSubcoreMesh(core_axis_name="c", subcore_axis_name="s")
    run = pl.kernel(
        functools.partial(_sc_body, B, T, S, D, TPW, TC),
        out_type=jax.ShapeDtypeStruct((B, T, S, D), jnp.float32),
        mesh=mesh,
        scratch_types=(
            [pltpu.VMEM((TC, S, D), jnp.float32) for _ in range(_NB)]
            + [pltpu.VMEM((TC, D), jnp.float32) for _ in range(_NB)]
            + [pltpu.SemaphoreType.DMA for _ in range(3 * _NB)]
        ),
    )
    return run(x, pos_embedding)


# speedup vs baseline: 1.0120x; 1.0120x over previous
import functools

import jax
import jax.numpy as jnp
from jax import lax
from jax.experimental import pallas as pl
from jax.experimental.pallas import tpu as pltpu
from jax.experimental.pallas import tpu_sc as plsc

_L = 16


def _sc_body(B, T, S, D, TPW, TC, x_hbm, emb_hbm, out_hbm,
             xb0, xb1, eb0, eb1, sx0, sx1, se0, se1, so0, so1):
    wid = lax.axis_index("s") * 2 + lax.axis_index("c")
    t0 = wid * TPW
    cpb = TPW // TC
    nch = B * cpb
    shift = cpb.bit_length() - 1
    bufs = ((xb0, eb0, sx0, se0, so0), (xb1, eb1, sx1, se1, so1))

    def coords(i):
        b = lax.shift_right_logical(i, shift)
        c = lax.bitwise_and(i, cpb - 1)
        return b, t0 + c * TC

    def start_in(i, slot):
        xb, eb, sx, se, _ = bufs[slot]
        b, tb = coords(i)
        pltpu.make_async_copy(x_hbm.at[b, pl.ds(tb, TC)], xb, sx).start()
        pltpu.make_async_copy(emb_hbm.at[pl.ds(tb, TC)], eb, se).start()

    def wait_in(slot):
        xb, eb, sx, se, _ = bufs[slot]
        pltpu.make_async_copy(x_hbm.at[0, pl.ds(0, TC)], xb, sx).wait()
        pltpu.make_async_copy(emb_hbm.at[pl.ds(0, TC)], eb, se).wait()

    def start_out(i, slot):
        xb, _, _, _, so = bufs[slot]
        b, tb = coords(i)
        pltpu.make_async_copy(xb, out_hbm.at[b, pl.ds(tb, TC)], so).start()

    def wait_out(slot):
        xb, _, _, _, so = bufs[slot]
        pltpu.make_async_copy(xb, out_hbm.at[0, pl.ds(0, TC)], so).wait()

    def compute(slot):
        xb, eb = bufs[slot][0], bufs[slot][1]
        for t in range(TC):
            def body(k2, carry):
                for u in range(2):
                    sl = pl.ds(k2 * 2 * _L + u * _L, _L)
                    ev = eb[t, sl]
                    for s_ in range(S):
                        xb[t, s_, sl] = xb[t, s_, sl] + ev
                return carry
            lax.fori_loop(0, D // (2 * _L), body, 0)

    start_in(jnp.int32(0), 0)

    def pair(i2, carry):
        for j in (0, 1):
            i = i2 * 2 + j

            @pl.when(i + 1 < nch)
            def _prefetch():
                @pl.when(i >= 1)
                def _drain():
                    wait_out(1 - j)
                start_in(i + 1, 1 - j)

            wait_in(j)
            compute(j)
            start_out(i, j)
        return carry

    lax.fori_loop(0, nch // 2, pair, 0)
    wait_out(0)
    wait_out(1)


def kernel(x, pos_embedding):
    B, T, S, D = x.shape
    NW = 32
    TPW = T // NW
    TC = 8

    mesh = plsc.VectorSubcoreMesh(core_axis_name="c", subcore_axis_name="s")
    run = pl.kernel(
        functools.partial(_sc_body, B, T, S, D, TPW, TC),
        out_type=jax.ShapeDtypeStruct((B, T, S, D), jnp.float32),
        mesh=mesh,
        scratch_types=[
            pltpu.VMEM((TC, S, D), jnp.float32),
            pltpu.VMEM((TC, S, D), jnp.float32),
            pltpu.VMEM((TC, D), jnp.float32),
            pltpu.VMEM((TC, D), jnp.float32),
            pltpu.SemaphoreType.DMA,
            pltpu.SemaphoreType.DMA,
            pltpu.SemaphoreType.DMA,
            pltpu.SemaphoreType.DMA,
            pltpu.SemaphoreType.DMA,
            pltpu.SemaphoreType.DMA,
        ],
    )
    return run(x, pos_embedding)


# speedup vs baseline: 2.0932x; 2.0683x over previous
import functools

import jax
import jax.numpy as jnp
from jax import lax
from jax.experimental import pallas as pl
from jax.experimental.pallas import tpu as pltpu
from jax.experimental.pallas import tpu_sc as plsc

_L = 16
_NB = 3


def _sc_body(B, T, S, D, TPW, TC, x_hbm, emb_hbm, out_hbm, *scratch):
    wid = lax.axis_index("s") * 2 + lax.axis_index("c")
    t0 = wid * TPW
    cpb = TPW // TC
    nch = B * cpb
    shift = cpb.bit_length() - 1
    xbs, ebs = scratch[0:_NB], scratch[_NB:2 * _NB]
    sxs, ses, sos = (scratch[2 * _NB:3 * _NB], scratch[3 * _NB:4 * _NB],
                     scratch[4 * _NB:5 * _NB])

    def coords(i):
        b = lax.shift_right_logical(i, shift)
        c = lax.bitwise_and(i, cpb - 1)
        return b, t0 + c * TC

    def start_in(i, slot):
        b, tb = coords(i)
        pltpu.make_async_copy(
            x_hbm.at[b, pl.ds(tb, TC)], xbs[slot], sxs[slot]).start()
        pltpu.make_async_copy(
            emb_hbm.at[pl.ds(tb, TC)], ebs[slot], ses[slot]).start()

    def wait_in(slot):
        pltpu.make_async_copy(
            x_hbm.at[0, pl.ds(0, TC)], xbs[slot], sxs[slot]).wait()
        pltpu.make_async_copy(
            emb_hbm.at[pl.ds(0, TC)], ebs[slot], ses[slot]).wait()

    def start_out(i, slot):
        b, tb = coords(i)
        pltpu.make_async_copy(
            xbs[slot], out_hbm.at[b, pl.ds(tb, TC)], sos[slot]).start()

    def wait_out(slot):
        pltpu.make_async_copy(
            xbs[slot], out_hbm.at[0, pl.ds(0, TC)], sos[slot]).wait()

    def compute(slot):
        xb, eb = xbs[slot], ebs[slot]
        for t in range(TC):
            def body(k, carry):
                ev = eb[t, pl.ds(k * _L, _L)]
                for s_ in range(S):
                    xb[t, s_, pl.ds(k * _L, _L)] = (
                        xb[t, s_, pl.ds(k * _L, _L)] + ev)
                return carry
            lax.fori_loop(0, D // _L, body, 0)

    for j in range(_NB - 1):
        start_in(jnp.int32(j), j)

    n_rings = (nch - (_NB - 1)) // _NB
    assert n_rings * _NB + (_NB - 1) == nch

    def ring(iq, carry):
        for j in range(_NB):
            i = iq * _NB + j
            tgt = (j + _NB - 1) % _NB

            @pl.when(i >= 1)
            def _drain():
                wait_out(tgt)

            start_in(i + _NB - 1, tgt)
            wait_in(j)
            compute(j)
            start_out(i, j)
        return carry

    lax.fori_loop(0, n_rings, ring, 0)

    for r in range(_NB - 1):
        i = n_rings * _NB + r
        j = i % _NB
        wait_in(j)
        compute(j)
        start_out(jnp.int32(i), j)

    for j in range(_NB):
        wait_out(j)


def kernel(x, pos_embedding):
    B, T, S, D = x.shape
    NW = 32
    TPW = T // NW
    TC = 8

    mesh = plsc.VectorSubcoreMesh(core_axis_name="c", subcore_axis_name="s")
    run = pl.kernel(
        functools.partial(_sc_body, B, T, S, D, TPW, TC),
        out_type=jax.ShapeDtypeStruct((B, T, S, D), jnp.float32),
        mesh=mesh,
        scratch_types=(
            [pltpu.VMEM((TC, S, D), jnp.float32) for _ in range(_NB)]
            + [pltpu.VMEM((TC, D), jnp.float32) for _ in range(_NB)]
            + [pltpu.SemaphoreType.DMA for _ in range(3 * _NB)]
        ),
    )
    return run(x, pos_embedding)


# speedup vs baseline: 2.8522x; 1.3627x over previous
import functools

import jax
import jax.numpy as jnp
from jax import lax
from jax.experimental import pallas as pl
from jax.experimental.pallas import tpu as pltpu
from jax.experimental.pallas import tpu_sc as plsc

_L = 16


def _sc_body(B, T, S, D, TPW, TC, x_hbm, emb_hbm, out_hbm,
             xb0, xb1, eb0, eb1, sx0, sx1, se0, se1, so0, so1):
    wid = lax.axis_index("s") * 2 + lax.axis_index("c")
    t0 = wid * TPW
    cpb = TPW // TC
    nch = B * cpb
    shift = cpb.bit_length() - 1
    bufs = ((xb0, eb0, sx0, se0, so0), (xb1, eb1, sx1, se1, so1))

    def coords(i):
        b = lax.shift_right_logical(i, shift)
        c = lax.bitwise_and(i, cpb - 1)
        return b, t0 + c * TC

    def start_in(i, slot):
        xb, eb, sx, se, _ = bufs[slot]
        b, tb = coords(i)
        pltpu.make_async_copy(x_hbm.at[b, pl.ds(tb, TC)], xb, sx).start()
        pltpu.make_async_copy(emb_hbm.at[pl.ds(tb, TC)], eb, se).start()

    def wait_in(slot):
        xb, eb, sx, se, _ = bufs[slot]
        pltpu.make_async_copy(x_hbm.at[0, pl.ds(0, TC)], xb, sx).wait()
        pltpu.make_async_copy(emb_hbm.at[pl.ds(0, TC)], eb, se).wait()

    def start_out(i, slot):
        xb, _, _, _, so = bufs[slot]
        b, tb = coords(i)
        pltpu.make_async_copy(xb, out_hbm.at[b, pl.ds(tb, TC)], so).start()

    def wait_out(slot):
        xb, _, _, _, so = bufs[slot]
        pltpu.make_async_copy(xb, out_hbm.at[0, pl.ds(0, TC)], so).wait()

    def compute(slot):
        xb, eb = bufs[slot][0], bufs[slot][1]
        for t in range(TC):
            @plsc.parallel_loop(0, D // _L, unroll=4)
            def _body(k):
                ev = eb[t, pl.ds(k * _L, _L)]
                for s_ in range(S):
                    xb[t, s_, pl.ds(k * _L, _L)] = (
                        xb[t, s_, pl.ds(k * _L, _L)] + ev)

    start_in(jnp.int32(0), 0)

    def pair(i2, carry):
        for j in (0, 1):
            i = i2 * 2 + j

            @pl.when(i + 1 < nch)
            def _prefetch():
                @pl.when(i >= 1)
                def _drain():
                    wait_out(1 - j)
                start_in(i + 1, 1 - j)

            wait_in(j)
            compute(j)
            start_out(i, j)
        return carry

    lax.fori_loop(0, nch // 2, pair, 0)
    wait_out(0)
    wait_out(1)


def kernel(x, pos_embedding):
    B, T, S, D = x.shape
    NW = 32
    TPW = T // NW
    TC = 8

    mesh = plsc.VectorSubcoreMesh(core_axis_name="c", subcore_axis_name="s")
    run = pl.kernel(
        functools.partial(_sc_body, B, T, S, D, TPW, TC),
        out_type=jax.ShapeDtypeStruct((B, T, S, D), jnp.float32),
        mesh=mesh,
        scratch_types=[
            pltpu.VMEM((TC, S, D), jnp.float32),
            pltpu.VMEM((TC, S, D), jnp.float32),
            pltpu.VMEM((TC, D), jnp.float32),
            pltpu.VMEM((TC, D), jnp.float32),
            pltpu.SemaphoreType.DMA,
            pltpu.SemaphoreType.DMA,
            pltpu.SemaphoreType.DMA,
            pltpu.SemaphoreType.DMA,
            pltpu.SemaphoreType.DMA,
            pltpu.SemaphoreType.DMA,
        ],
    )
    return run(x, pos_embedding)


# speedup vs baseline: 2.8921x; 1.0140x over previous
import functools

import jax
import jax.numpy as jnp
from jax import lax
from jax.experimental import pallas as pl
from jax.experimental.pallas import tpu as pltpu
from jax.experimental.pallas import tpu_sc as plsc

_L = 16


def _sc_body(B, T, S, D, TPW, TC, x_hbm, emb_hbm, out_hbm,
             xb0, xb1, eb0, eb1, sx0, sx1, se0, se1, so0, so1):
    wid = lax.axis_index("s") * 2 + lax.axis_index("c")
    t0 = wid * TPW
    cpb = TPW // TC
    nch = B * cpb
    shift = cpb.bit_length() - 1
    bufs = ((xb0, eb0, sx0, se0, so0), (xb1, eb1, sx1, se1, so1))

    def coords(i):
        b = lax.shift_right_logical(i, shift)
        c = lax.bitwise_and(i, cpb - 1)
        return b, t0 + c * TC

    def start_in(i, slot):
        xb, eb, sx, se, _ = bufs[slot]
        b, tb = coords(i)
        pltpu.make_async_copy(x_hbm.at[b, pl.ds(tb, TC)], xb, sx).start()
        pltpu.make_async_copy(emb_hbm.at[pl.ds(tb, TC)], eb, se).start()

    def wait_in(slot):
        xb, eb, sx, se, _ = bufs[slot]
        pltpu.make_async_copy(x_hbm.at[0, pl.ds(0, TC)], xb, sx).wait()
        pltpu.make_async_copy(emb_hbm.at[pl.ds(0, TC)], eb, se).wait()

    def start_out(i, slot):
        xb, _, _, _, so = bufs[slot]
        b, tb = coords(i)
        pltpu.make_async_copy(xb, out_hbm.at[b, pl.ds(tb, TC)], so).start()

    def wait_out(slot):
        xb, _, _, _, so = bufs[slot]
        pltpu.make_async_copy(xb, out_hbm.at[0, pl.ds(0, TC)], so).wait()

    def compute(slot):
        xb, eb = bufs[slot][0], bufs[slot][1]
        for t in range(TC):
            @plsc.parallel_loop(0, D // _L, unroll=8)
            def _body(k):
                ev = eb[t, pl.ds(k * _L, _L)]
                for s_ in range(S):
                    xb[t, s_, pl.ds(k * _L, _L)] = (
                        xb[t, s_, pl.ds(k * _L, _L)] + ev)

    start_in(jnp.int32(0), 0)

    def pair(i2, carry):
        for j in (0, 1):
            i = i2 * 2 + j

            @pl.when(i + 1 < nch)
            def _prefetch():
                @pl.when(i >= 1)
                def _drain():
                    wait_out(1 - j)
                start_in(i + 1, 1 - j)

            wait_in(j)
            compute(j)
            start_out(i, j)
        return carry

    lax.fori_loop(0, nch // 2, pair, 0)
    wait_out(0)
    wait_out(1)


def kernel(x, pos_embedding):
    B, T, S, D = x.shape
    NW = 32
    TPW = T // NW
    TC = 8

    mesh = plsc.VectorSubcoreMesh(core_axis_name="c", subcore_axis_name="s")
    run = pl.kernel(
        functools.partial(_sc_body, B, T, S, D, TPW, TC),
        out_type=jax.ShapeDtypeStruct((B, T, S, D), jnp.float32),
        mesh=mesh,
        scratch_types=[
            pltpu.VMEM((TC, S, D), jnp.float32),
            pltpu.VMEM((TC, S, D), jnp.float32),
            pltpu.VMEM((TC, D), jnp.float32),
            pltpu.VMEM((TC, D), jnp.float32),
            pltpu.SemaphoreType.DMA,
            pltpu.SemaphoreType.DMA,
            pltpu.SemaphoreType.DMA,
            pltpu.SemaphoreType.DMA,
            pltpu.SemaphoreType.DMA,
            pltpu.SemaphoreType.DMA,
        ],
    )
    return run(x, pos_embedding)
